# Initial kernel scaffold; baseline (speedup 1.0000x reference)
import jax
import jax.numpy as jnp
from jax.experimental import pallas as pl


def kernel(x, edge_index, W1, b1, W2, b2, Wp1, bp1, Wp2, bp2, Wc, bc):
    raise NotImplementedError("write your pallas kernel here")


# speedup vs baseline: 9.1484x; 9.1484x over previous
import functools

import jax
import jax.numpy as jnp
from jax import lax
from jax.experimental import pallas as pl
from jax.experimental.pallas import tpu as pltpu
from jax.experimental.pallas import tpu_sc as plsc

NN = 10000
EE = 320000
DD = 128
NW = 32
EPT = EE // NW
KC = 80
CH = EPT // KC
NP = 10240
RPT = NP // 16

_mesh = plsc.VectorSubcoreMesh(core_axis_name="c", subcore_axis_name="s")


def _sc_degree(dst_r, ones16, z16):
    @functools.partial(
        pl.kernel, mesh=_mesh,
        out_type=jax.ShapeDtypeStruct((2, NP, 16), jnp.float32),
        scratch_types=[
            pltpu.VMEM((CH, KC), jnp.int32),
            pltpu.VMEM((KC, 16), jnp.float32),
            pltpu.VMEM_SHARED((NP, 16), jnp.float32),
            pltpu.SemaphoreType.DMA,
        ],
    )
    def k(dst_hbm, ones_hbm, z_hbm, out_hbm, didx_v, ones_v, acc_sh, sem):
        cid = lax.axis_index("c")
        sid = lax.axis_index("s")
        wid = sid * 2 + cid
        pltpu.sync_copy(z_hbm, acc_sh.at[pl.ds(sid * RPT, RPT)])
        pltpu.sync_copy(dst_hbm.at[wid], didx_v)
        pltpu.sync_copy(ones_hbm, ones_v)
        plsc.subcore_barrier()

        def body(j, c):
            pltpu.sync_copy(ones_v, acc_sh.at[didx_v.at[j]], add=True)
            return c

        lax.fori_loop(0, CH, body, 0)
        plsc.subcore_barrier()
        pltpu.sync_copy(acc_sh.at[pl.ds(sid * RPT, RPT)],
                        out_hbm.at[cid, pl.ds(sid * RPT, RPT)])

    return k(dst_r, ones16, z16)


def _sc_aggregate(g, src_r, dst_r, z128):
    @functools.partial(
        pl.kernel, mesh=_mesh,
        out_type=jax.ShapeDtypeStruct((2, NP, DD), jnp.float32),
        scratch_types=[
            pltpu.VMEM((CH, KC), jnp.int32),
            pltpu.VMEM((CH, KC), jnp.int32),
            pltpu.VMEM((KC, DD), jnp.float32),
            pltpu.VMEM_SHARED((NP, DD), jnp.float32),
            pltpu.SemaphoreType.DMA,
        ],
    )
    def k(g_hbm, src_hbm, dst_hbm, z_hbm, out_hbm,
          sidx_v, didx_v, rows_v, acc_sh, sem):
        cid = lax.axis_index("c")
        sid = lax.axis_index("s")
        wid = sid * 2 + cid
        pltpu.sync_copy(z_hbm, acc_sh.at[pl.ds(sid * RPT, RPT)])
        pltpu.sync_copy(src_hbm.at[wid], sidx_v)
        pltpu.sync_copy(dst_hbm.at[wid], didx_v)
        plsc.subcore_barrier()

        def body(j, c):
            pltpu.async_copy(g_hbm.at[sidx_v.at[j]], rows_v, sem).wait()
            pltpu.sync_copy(rows_v, acc_sh.at[didx_v.at[j]], add=True)
            return c

        lax.fori_loop(0, CH, body, 0)
        plsc.subcore_barrier()
        pltpu.sync_copy(acc_sh.at[pl.ds(sid * RPT, RPT)],
                        out_hbm.at[cid, pl.ds(sid * RPT, RPT)])

    return k(g, src_r, dst_r, z128)


def _sc_pair_gather(f, src_r, dst_r):
    @functools.partial(
        pl.kernel, mesh=_mesh,
        out_type=(jax.ShapeDtypeStruct((EE, DD), jnp.float32),
                  jax.ShapeDtypeStruct((EE, DD), jnp.float32)),
        scratch_types=[
            pltpu.VMEM((CH, KC), jnp.int32),
            pltpu.VMEM((CH, KC), jnp.int32),
            pltpu.VMEM((KC, DD), jnp.float32),
            pltpu.VMEM((KC, DD), jnp.float32),
            pltpu.SemaphoreType.DMA,
            pltpu.SemaphoreType.DMA,
        ],
    )
    def k(f_hbm, src_hbm, dst_hbm, os_hbm, od_hbm,
          sidx_v, didx_v, rs_v, rd_v, sem_s, sem_d):
        cid = lax.axis_index("c")
        sid = lax.axis_index("s")
        wid = sid * 2 + cid
        base0 = wid * EPT
        pltpu.sync_copy(src_hbm.at[wid], sidx_v)
        pltpu.sync_copy(dst_hbm.at[wid], didx_v)

        def body(j, c):
            cs = pltpu.async_copy(f_hbm.at[sidx_v.at[j]], rs_v, sem_s)
            cd = pltpu.async_copy(f_hbm.at[didx_v.at[j]], rd_v, sem_d)
            base = pl.multiple_of(base0 + j * KC, 8)
            cs.wait()
            pltpu.sync_copy(rs_v, os_hbm.at[pl.ds(base, KC)])
            cd.wait()
            pltpu.sync_copy(rd_v, od_hbm.at[pl.ds(base, KC)])
            return c

        lax.fori_loop(0, CH, body, 0)

    return k(f, src_r, dst_r)


def _dinv_col(dp_ref):
    deg = dp_ref[0] + dp_ref[1] + 1.0
    dinv = lax.rsqrt(deg)
    return dinv[:, 0:1]


def _tc_g1(x, W1, deg_p):
    R = 1000

    def body(x_ref, w_ref, dp_ref, o_ref):
        col = _dinv_col(dp_ref)
        h = jnp.dot(x_ref[...], w_ref[...], preferred_element_type=jnp.float32)
        o_ref[...] = h * col

    return pl.pallas_call(
        body,
        grid=(NN // R,),
        in_specs=[
            pl.BlockSpec((R, DD), lambda i: (i, 0)),
            pl.BlockSpec((DD, DD), lambda i: (0, 0)),
            pl.BlockSpec((2, R, 16), lambda i: (0, i, 0)),
        ],
        out_specs=pl.BlockSpec((R, DD), lambda i: (i, 0)),
        out_shape=jax.ShapeDtypeStruct((NN, DD), jnp.float32),
    )(x, W1, deg_p)


def _tc_mid(S_p, g1, deg_p, W2, b1):
    R = 1000

    def body(sp_ref, g_ref, dp_ref, w_ref, b_ref, o_ref):
        col = _dinv_col(dp_ref)
        x1 = (sp_ref[0] + sp_ref[1] + g_ref[...]) * col + b_ref[...]
        x1 = jnp.maximum(x1, 0.0)
        h = jnp.dot(x1, w_ref[...], preferred_element_type=jnp.float32)
        o_ref[...] = h * col

    return pl.pallas_call(
        body,
        grid=(NN // R,),
        in_specs=[
            pl.BlockSpec((2, R, DD), lambda i: (0, i, 0)),
            pl.BlockSpec((R, DD), lambda i: (i, 0)),
            pl.BlockSpec((2, R, 16), lambda i: (0, i, 0)),
            pl.BlockSpec((DD, DD), lambda i: (0, 0)),
            pl.BlockSpec((1, DD), lambda i: (0, 0)),
        ],
        out_specs=pl.BlockSpec((R, DD), lambda i: (i, 0)),
        out_shape=jax.ShapeDtypeStruct((NN, DD), jnp.float32),
    )(S_p, g1, deg_p, W2, b1)


def _tc_final(S_p, g2, deg_p, b2, Wc, bc):
    R = 1000
    ncls = Wc.shape[1]

    def body(sp_ref, g_ref, dp_ref, b_ref, wc_ref, bc_ref, f_ref, lg_ref):
        col = _dinv_col(dp_ref)
        f = (sp_ref[0] + sp_ref[1] + g_ref[...]) * col + b_ref[...]
        f_ref[...] = f
        lg_ref[...] = jnp.dot(f, wc_ref[...],
                              preferred_element_type=jnp.float32) + bc_ref[...]

    return pl.pallas_call(
        body,
        grid=(NN // R,),
        in_specs=[
            pl.BlockSpec((2, R, DD), lambda i: (0, i, 0)),
            pl.BlockSpec((R, DD), lambda i: (i, 0)),
            pl.BlockSpec((2, R, 16), lambda i: (0, i, 0)),
            pl.BlockSpec((1, DD), lambda i: (0, 0)),
            pl.BlockSpec((DD, ncls), lambda i: (0, 0)),
            pl.BlockSpec((1, ncls), lambda i: (0, 0)),
        ],
        out_specs=[
            pl.BlockSpec((R, DD), lambda i: (i, 0)),
            pl.BlockSpec((R, ncls), lambda i: (i, 0)),
        ],
        out_shape=[
            jax.ShapeDtypeStruct((NN, DD), jnp.float32),
            jax.ShapeDtypeStruct((NN, ncls), jnp.float32),
        ],
    )(S_p, g2, deg_p, b2, Wc, bc)


def _tc_edge_mlp(fs, fd, Wa, Wb, bp1, Wp2, bp2):
    BK = 1000
    DH = Wa.shape[1]

    def body(fs_ref, fd_ref, a_ref, b_ref, b1_ref, w2_ref, b2_ref, o_ref):
        h = jnp.dot(fs_ref[...], a_ref[...], preferred_element_type=jnp.float32)
        h = h + jnp.dot(fd_ref[...], b_ref[...],
                        preferred_element_type=jnp.float32)
        h = jnp.maximum(h + b1_ref[...], 0.0)
        o_ref[...] = jnp.dot(h, w2_ref[...],
                             preferred_element_type=jnp.float32) + b2_ref[...]

    return pl.pallas_call(
        body,
        grid=(EE // BK,),
        in_specs=[
            pl.BlockSpec((BK, DD), lambda i: (i, 0)),
            pl.BlockSpec((BK, DD), lambda i: (i, 0)),
            pl.BlockSpec((DD, DH), lambda i: (0, 0)),
            pl.BlockSpec((DD, DH), lambda i: (0, 0)),
            pl.BlockSpec((1, DH), lambda i: (0, 0)),
            pl.BlockSpec((DH, DD), lambda i: (0, 0)),
            pl.BlockSpec((1, DD), lambda i: (0, 0)),
        ],
        out_specs=pl.BlockSpec((BK, DD), lambda i: (i, 0)),
        out_shape=jax.ShapeDtypeStruct((EE, DD), jnp.float32),
    )(fs, fd, Wa, Wb, bp1, Wp2, bp2)


def kernel(x, edge_index, W1, b1, W2, b2, Wp1, bp1, Wp2, bp2, Wc, bc):
    src_r = edge_index[0].reshape(NW, CH, KC)
    dst_r = edge_index[1].reshape(NW, CH, KC)
    ones16 = jnp.ones((KC, 16), jnp.float32)
    z16 = jnp.zeros((RPT, 16), jnp.float32)
    z128 = jnp.zeros((RPT, DD), jnp.float32)

    deg_p = _sc_degree(dst_r, ones16, z16)
    g1 = _tc_g1(x, W1, deg_p)
    S1 = _sc_aggregate(g1, src_r, dst_r, z128)
    g2 = _tc_mid(S1, g1, deg_p, W2, b1.reshape(1, -1))
    S2 = _sc_aggregate(g2, src_r, dst_r, z128)
    f, logits = _tc_final(S2, g2, deg_p, b2.reshape(1, -1), Wc, bc.reshape(1, -1))
    fs, fd = _sc_pair_gather(f, src_r, dst_r)
    edge_feats = _tc_edge_mlp(fs, fd, Wp1[:DD], Wp1[DD:], bp1.reshape(1, -1),
                              Wp2, bp2.reshape(1, -1))
    return (f, edge_feats, logits, edge_index)
